# baseline (speedup 1.0000x reference)
import functools

import jax
import jax.numpy as jnp
from jax import lax
from jax.experimental import pallas as pl
from jax.experimental.pallas import tpu as pltpu
from jax.experimental.pallas import tpu_sc as plsc

_NB = 4096
_NT = 200
_D = 64
_DP = 128
_V = 100001

_NC = 2
_NS = 16
_NW = _NC * _NS

_BC = 128
_UPT = _NB // _BC
_UNITS = _NT * _UPT
_PER_W = _UNITS // _NW
_GROUPS = _PER_W // 2


def _emb_kernel(table_hbm, idxt_hbm, out_hbm, idx_v, rows_v, stage_v, in_sems, g_sems, out_sems):
    wid = lax.axis_index("s") * _NC + lax.axis_index("c")
    u0 = wid * _PER_W

    iota = lax.iota(jnp.int32, 16)
    d_idx = [iota + (16 * q) for q in range(4)]

    def group(g, carry):
        @pl.when(g > 0)
        def _():
            for p in range(2):
                pltpu.make_async_copy(
                    stage_v.at[p, :, pl.ds(0, _BC)], out_hbm.at[0, :, pl.ds(0, _BC)], out_sems.at[p]
                ).wait()

        us = [u0 + 2 * g, u0 + 2 * g + 1]
        ts = [u // _UPT for u in us]
        bs = [(u % _UPT) * _BC for u in us]

        idx_handles = []
        for p in range(2):
            idx_handles.append(
                pltpu.async_copy(
                    idxt_hbm.at[ts[p], pl.ds(bs[p], _BC)], idx_v.at[p], in_sems.at[p]
                )
            )
        g_handles = []
        for p in range(2):
            idx_handles[p].wait()
            g_handles.append(
                pltpu.async_copy(table_hbm.at[idx_v.at[p]], rows_v.at[p], g_sems.at[p])
            )
        for p in range(2):
            g_handles[p].wait()

            def body_b(b, c, p=p):
                b_splat = jnp.full((16,), b, jnp.int32)
                for q in range(4):
                    vals = rows_v[p, b, pl.ds(16 * q, 16)]
                    plsc.store_scatter(stage_v.at[p], [d_idx[q], b_splat], vals)
                return c

            lax.fori_loop(0, _BC, body_b, 0, unroll=16)
            pltpu.async_copy(
                stage_v.at[p, :, pl.ds(0, _BC)], out_hbm.at[ts[p], :, pl.ds(bs[p], _BC)], out_sems.at[p]
            )
        return carry

    lax.fori_loop(0, _GROUPS, group, 0)

    for p in range(2):
        pltpu.make_async_copy(
            stage_v.at[p, :, pl.ds(0, _BC)], out_hbm.at[0, :, pl.ds(0, _BC)], out_sems.at[p]
        ).wait()


@jax.jit
def _run(indices, table):
    idx_t = jnp.transpose(indices.astype(jnp.int32))
    table_p = jnp.pad(table, ((0, 0), (0, _DP - _D)))
    mesh = plsc.VectorSubcoreMesh(core_axis_name="c", subcore_axis_name="s")
    k = functools.partial(
        pl.kernel,
        out_type=jax.ShapeDtypeStruct((_NT, _D, _NB), jnp.float32),
        mesh=mesh,
        scratch_types=[
            pltpu.VMEM((2, _BC), jnp.int32),
            pltpu.VMEM((2, _BC, _DP), jnp.float32),
            pltpu.VMEM((2, _D, _BC + 9), jnp.float32),
            pltpu.SemaphoreType.DMA((2,)),
            pltpu.SemaphoreType.DMA((2,)),
            pltpu.SemaphoreType.DMA((2,)),
        ],
        compiler_params=pltpu.CompilerParams(needs_layout_passes=False),
    )(_emb_kernel)
    out3 = k(table_p, idx_t)
    return jnp.transpose(out3, (2, 0, 1))


def kernel(indices, table):
    return _run(indices, table)


# speedup vs baseline: 1.1024x; 1.1024x over previous
import functools

import jax
import jax.numpy as jnp
from jax import lax
from jax.experimental import pallas as pl
from jax.experimental.pallas import tpu as pltpu
from jax.experimental.pallas import tpu_sc as plsc

_NB = 4096
_NT = 200
_D = 64
_DP = 128
_V = 100001

_NC = 2
_NS = 16
_NW = _NC * _NS

_BC = 128
_UPT = _NB // _BC
_UNITS = _NT * _UPT
_PER_W = _UNITS // _NW
_GROUPS = _PER_W // 2


def _emb_kernel(table_hbm, idxt_hbm, out_hbm, idx_v, rows_v, stage_v, in_sems, g_sems, out_sems):
    wid = lax.axis_index("s") * _NC + lax.axis_index("c")
    u0 = wid * _PER_W

    iota = lax.iota(jnp.int32, 16)
    d_idx = [iota + (16 * q) for q in range(4)]

    def group(g, carry):
        @pl.when(g > 0)
        def _():
            for p in range(2):
                pltpu.make_async_copy(
                    stage_v.at[p, :, pl.ds(0, _BC)], out_hbm.at[0, :, pl.ds(0, _BC)], out_sems.at[p]
                ).wait()

        us = [u0 + 2 * g, u0 + 2 * g + 1]
        ts = [u // _UPT for u in us]
        bs = [(u % _UPT) * _BC for u in us]

        idx_handles = []
        for p in range(2):
            idx_handles.append(
                pltpu.async_copy(
                    idxt_hbm.at[ts[p], pl.ds(bs[p], _BC)], idx_v.at[p], in_sems.at[p]
                )
            )
        g_handles = []
        for p in range(2):
            idx_handles[p].wait()
            g_handles.append(
                pltpu.async_copy(table_hbm.at[idx_v.at[p]], rows_v.at[p], g_sems.at[p])
            )
        for p in range(2):
            g_handles[p].wait()

            @plsc.parallel_loop(0, _BC, step=1, unroll=8)
            def _transpose(b, p=p):
                b_splat = jnp.full((16,), b, jnp.int32)
                vs = [rows_v[p, b, pl.ds(16 * q, 16)] for q in range(4)]
                for q in range(4):
                    plsc.store_scatter(stage_v.at[p], [d_idx[q], b_splat], vs[q])
            pltpu.async_copy(
                stage_v.at[p, :, pl.ds(0, _BC)], out_hbm.at[ts[p], :, pl.ds(bs[p], _BC)], out_sems.at[p]
            )
        return carry

    lax.fori_loop(0, _GROUPS, group, 0)

    for p in range(2):
        pltpu.make_async_copy(
            stage_v.at[p, :, pl.ds(0, _BC)], out_hbm.at[0, :, pl.ds(0, _BC)], out_sems.at[p]
        ).wait()


@jax.jit
def _run(indices, table):
    idx_t = jnp.transpose(indices.astype(jnp.int32))
    table_p = jnp.pad(table, ((0, 0), (0, _DP - _D)))
    mesh = plsc.VectorSubcoreMesh(core_axis_name="c", subcore_axis_name="s")
    k = functools.partial(
        pl.kernel,
        out_type=jax.ShapeDtypeStruct((_NT, _D, _NB), jnp.float32),
        mesh=mesh,
        scratch_types=[
            pltpu.VMEM((2, _BC), jnp.int32),
            pltpu.VMEM((2, _BC, _DP), jnp.float32),
            pltpu.VMEM((2, _D, _BC + 9), jnp.float32),
            pltpu.SemaphoreType.DMA((2,)),
            pltpu.SemaphoreType.DMA((2,)),
            pltpu.SemaphoreType.DMA((2,)),
        ],
        compiler_params=pltpu.CompilerParams(needs_layout_passes=False),
    )(_emb_kernel)
    out3 = k(table_p, idx_t)
    return jnp.transpose(out3, (2, 0, 1))


def kernel(indices, table):
    return _run(indices, table)


# speedup vs baseline: 2.0041x; 1.8180x over previous
import functools

import jax
import jax.numpy as jnp
from jax import lax
from jax.experimental import pallas as pl
from jax.experimental.pallas import tpu as pltpu
from jax.experimental.pallas import tpu_sc as plsc

_BATCH = 4096
_HIST = 200
_D = 64
_B = _BATCH * _HIST

_NC = 2
_NS = 16
_NW = _NC * _NS
_B_PER_W = _B // _NW

_CHUNK = 512
_NBUF = 2
assert _B_PER_W % (_CHUNK * _NBUF) == 0

_M = _BATCH
_N = _HIST * _D
_BM = 512
_BN = 512


def _gather_kernel(table_hbm, idx_hbm, out_hbm, idx_v, rows_v, in_sems, g_sems, out_sems):
    wid = lax.axis_index("s") * _NC + lax.axis_index("c")
    base = wid * _B_PER_W
    ngroup = _B_PER_W // (_CHUNK * _NBUF)

    def group(g, carry):
        goff = base + g * (_CHUNK * _NBUF)

        @pl.when(g > 0)
        def _():
            for b in range(_NBUF):
                pltpu.make_async_copy(
                    rows_v.at[b], out_hbm.at[pl.ds(0, _CHUNK)], out_sems.at[b]
                ).wait()

        idx_handles = []
        for b in range(_NBUF):
            off = pl.multiple_of(goff + b * _CHUNK, _CHUNK)
            idx_handles.append(
                pltpu.async_copy(idx_hbm.at[pl.ds(off, _CHUNK)], idx_v.at[b], in_sems.at[b])
            )
        g_handles = []
        for b in range(_NBUF):
            idx_handles[b].wait()
            g_handles.append(
                pltpu.async_copy(table_hbm.at[idx_v.at[b]], rows_v.at[b], g_sems.at[b])
            )
        for b in range(_NBUF):
            off = pl.multiple_of(goff + b * _CHUNK, _CHUNK)
            g_handles[b].wait()
            pltpu.async_copy(rows_v.at[b], out_hbm.at[pl.ds(off, _CHUNK)], out_sems.at[b])
        return carry

    lax.fori_loop(0, ngroup, group, 0)

    for b in range(_NBUF):
        pltpu.make_async_copy(
            rows_v.at[b], out_hbm.at[pl.ds(0, _CHUNK)], out_sems.at[b]
        ).wait()


def _transpose_kernel(x_ref, o_ref):
    o_ref[...] = x_ref[...].T


@jax.jit
def _run(indices, table):
    idx_flat = indices.reshape(_B).astype(jnp.int32)
    mesh = plsc.VectorSubcoreMesh(core_axis_name="c", subcore_axis_name="s")
    gather = functools.partial(
        pl.kernel,
        out_type=jax.ShapeDtypeStruct((_B, _D), jnp.float32),
        mesh=mesh,
        scratch_types=[
            pltpu.VMEM((_NBUF, _CHUNK), jnp.int32),
            pltpu.VMEM((_NBUF, _CHUNK, _D), jnp.float32),
            pltpu.SemaphoreType.DMA((_NBUF,)),
            pltpu.SemaphoreType.DMA((_NBUF,)),
            pltpu.SemaphoreType.DMA((_NBUF,)),
        ],
        compiler_params=pltpu.CompilerParams(use_tc_tiling_on_sc=False),
    )(_gather_kernel)
    flat = gather(table, idx_flat)

    x2 = flat.reshape(_M, _N)
    out2 = pl.pallas_call(
        _transpose_kernel,
        grid=(_M // _BM, _N // _BN),
        in_specs=[pl.BlockSpec((_BM, _BN), lambda i, j: (i, j))],
        out_specs=pl.BlockSpec((_BN, _BM), lambda i, j: (j, i)),
        out_shape=jax.ShapeDtypeStruct((_N, _M), jnp.float32),
    )(x2)
    out3 = out2.reshape(_HIST, _D, _BATCH)
    return jnp.transpose(out3, (2, 0, 1))


def kernel(indices, table):
    return _run(indices, table)
